# baseline (speedup 1.0000x reference)
import functools

import jax
import jax.numpy as jnp
from jax import lax
from jax.experimental import pallas as pl
from jax.experimental.pallas import tpu as pltpu
from jax.experimental.pallas import tpu_sc as plsc

_N = 10000
_E = 320000
_D = 128
_H = 64
_C = 40

_NC = 2
_NS = 16
_NW = _NC * _NS

_EPT = _E // _NW
_BLK = 80
_NBLK = _EPT // _BLK
_GRP = _BLK // 16

_STRIPE = 640
_LAST = _N - 15 * _STRIPE

_BR = 1000
_NP = _NS * _STRIPE


def _dense_in_body(beta_ref, x_ref, w_ref, b_ref, u_ref, ud_ref, nrm_ref):
    h = jnp.dot(x_ref[...], w_ref[...], preferred_element_type=jnp.float32)
    h = jnp.maximum(h + b_ref[...], 0.0)
    nrm = jnp.sqrt(jnp.sum(h * h, axis=1, keepdims=True))
    u = h / jnp.maximum(nrm, 1e-12)
    u_ref[...] = u
    ud_ref[...] = beta_ref[0, 0] * u
    nrm_ref[...] = nrm


def _dense_in(x, w1, b1, beta1):
    return pl.pallas_call(
        _dense_in_body,
        grid=(_N // _BR,),
        in_specs=[
            pl.BlockSpec((1, 1), lambda i: (0, 0)),
            pl.BlockSpec((_BR, _D), lambda i: (i, 0)),
            pl.BlockSpec((_D, _H), lambda i: (0, 0)),
            pl.BlockSpec((1, _H), lambda i: (0, 0)),
        ],
        out_specs=[
            pl.BlockSpec((_BR, _H), lambda i: (i, 0)),
            pl.BlockSpec((_BR, _H), lambda i: (i, 0)),
            pl.BlockSpec((_BR, 1), lambda i: (i, 0)),
        ],
        out_shape=[
            jax.ShapeDtypeStruct((_N, _H), jnp.float32),
            jax.ShapeDtypeStruct((_N, _H), jnp.float32),
            jax.ShapeDtypeStruct((_N, 1), jnp.float32),
        ],
    )(beta1.reshape(1, 1), x, w1, b1.reshape(1, _H))


def _combine_body(beta_ref, num_ref, den_ref, u_ref, ud_ref, nrm_ref):
    den = den_ref[0] + den_ref[1] + 1e-16
    h = (num_ref[0] + num_ref[1]) / den
    nrm = jnp.sqrt(jnp.sum(h * h, axis=1, keepdims=True))
    u = h / jnp.maximum(nrm, 1e-12)
    u_ref[...] = u
    ud_ref[...] = beta_ref[0, 0] * u
    nrm_ref[...] = nrm


def _combine(num, den, beta2):
    return pl.pallas_call(
        _combine_body,
        grid=(_N // _BR,),
        in_specs=[
            pl.BlockSpec((1, 1), lambda i: (0, 0)),
            pl.BlockSpec((_NC, _BR, _H), lambda i: (0, i, 0)),
            pl.BlockSpec((_NC, _BR, 1), lambda i: (0, i, 0)),
        ],
        out_specs=[
            pl.BlockSpec((_BR, _H), lambda i: (i, 0)),
            pl.BlockSpec((_BR, _H), lambda i: (i, 0)),
            pl.BlockSpec((_BR, 1), lambda i: (i, 0)),
        ],
        out_shape=[
            jax.ShapeDtypeStruct((_N, _H), jnp.float32),
            jax.ShapeDtypeStruct((_N, _H), jnp.float32),
            jax.ShapeDtypeStruct((_N, 1), jnp.float32),
        ],
    )(beta2.reshape(1, 1), num, den[:, 0, :_N].reshape(_NC, _N, 1))


def _final_body(num_ref, den_ref, w_ref, b_ref, out_ref):
    den = den_ref[0] + den_ref[1] + 1e-16
    h = (num_ref[0] + num_ref[1]) / den
    o = jnp.dot(h, w_ref[...], preferred_element_type=jnp.float32) + b_ref[...]
    m = jnp.max(o, axis=1, keepdims=True)
    lse = jnp.log(jnp.sum(jnp.exp(o - m), axis=1, keepdims=True))
    out_ref[...] = o - m - lse


def _final(num, den, w2, b2):
    return pl.pallas_call(
        _final_body,
        grid=(_N // _BR,),
        in_specs=[
            pl.BlockSpec((_NC, _BR, _H), lambda i: (0, i, 0)),
            pl.BlockSpec((_NC, _BR, 1), lambda i: (0, i, 0)),
            pl.BlockSpec((_H, _C), lambda i: (0, 0)),
            pl.BlockSpec((1, _C), lambda i: (0, 0)),
        ],
        out_specs=pl.BlockSpec((_BR, _C), lambda i: (i, 0)),
        out_shape=jax.ShapeDtypeStruct((_N, _C), jnp.float32),
    )(num, den[:, 0, :_N].reshape(_NC, _N, 1), w2, b2.reshape(1, _C))


def _compute_block(b, src_v, nrm_v, udst_v, usrc_v, ev_v):
    lanes = lax.iota(jnp.int32, 16)
    perms = [lanes ^ m for m in (1, 2, 4, 8)]
    gd = lax.GatherDimensionNumbers(offset_dims=(),
                                    collapsed_slice_dims=(0,),
                                    start_index_map=(0,))

    def pg(v, p):
        return lax.gather(v, p[:, None], gd, (1,),
                          mode=lax.GatherScatterMode.PROMISE_IN_BOUNDS)

    for g in range(_GRP):
        rr = pl.ds(g * 16, 16)
        sv = src_v[b, rr]
        nv = plsc.load_gather(nrm_v, [sv])
        evacc = jnp.zeros((16,), jnp.float32)
        for e in range(16):
            r = g * 16 + e
            d0 = udst_v[r, 0:16]
            d1 = udst_v[r, 16:32]
            d2 = udst_v[r, 32:48]
            d3 = udst_v[r, 48:64]
            s0 = usrc_v[r, 0:16]
            s1 = usrc_v[r, 16:32]
            s2 = usrc_v[r, 32:48]
            s3 = usrc_v[r, 48:64]
            a = d0 * s0 + d1 * s1 + d2 * s2 + d3 * s3
            for p in perms:
                a = a + pg(a, p)
            ev = jnp.exp(a)
            f = ev * pg(nv, jnp.full((16,), e, jnp.int32))
            usrc_v[r, 0:16] = s0 * f
            usrc_v[r, 16:32] = s1 * f
            usrc_v[r, 32:48] = s2 * f
            usrc_v[r, 48:64] = s3 * f
            evacc = jnp.where(lanes == e, ev, evacc)
        ev_v[rr] = evacc


def _edge_body(ud_hbm, u_hbm, nrm_hbm, src_hbm, dst_hbm, num_out, den_out,
               src_v, dst_v, udst0, usrc0, ev0, udst1, usrc1, ev1,
               udst2, usrc2, ev2, nrm_v, num_sh, den_sh,
               gsem0, gsem1, gsem2, ssem0, ssem1, ssem2):
    c = lax.axis_index("c")
    s = lax.axis_index("s")
    wid = c * _NS + s

    z16 = jnp.zeros((16,), jnp.float32)
    for r in range(_BLK):
        for k in range(4):
            udst0[r, k * 16:(k + 1) * 16] = z16
    for k in range(_BLK // 16):
        ev0[k * 16:(k + 1) * 16] = z16

    base = s * _STRIPE

    for k in range(_STRIPE // _BLK):
        pltpu.sync_copy(ev0, den_sh.at[pl.ds(base + k * _BLK, _BLK)])

    @pl.when(s < _NS - 1)
    def _():
        for k in range(_STRIPE // _BLK):
            pltpu.sync_copy(udst0, num_sh.at[pl.ds(base + k * _BLK, _BLK)])

    @pl.when(s == _NS - 1)
    def _():
        for k in range(_LAST // _BLK):
            pltpu.sync_copy(udst0, num_sh.at[pl.ds(base + k * _BLK, _BLK)])

    pltpu.sync_copy(src_hbm.at[wid], src_v)
    pltpu.sync_copy(dst_hbm.at[wid], dst_v)
    pltpu.sync_copy(nrm_hbm, nrm_v)

    plsc.subcore_barrier()

    def g_issue(b, udst_v, usrc_v, sem):
        pltpu.async_copy(ud_hbm.at[dst_v.at[b]], udst_v, sem)
        pltpu.async_copy(u_hbm.at[src_v.at[b]], usrc_v, sem)

    def g_wait(udst_v, usrc_v, sem):
        pltpu.make_async_copy(ud_hbm.at[pl.ds(0, _BLK)], udst_v, sem).wait()
        pltpu.make_async_copy(u_hbm.at[pl.ds(0, _BLK)], usrc_v, sem).wait()

    def s_issue(b, usrc_v, ev_buf, sem):
        pltpu.async_copy(usrc_v, num_sh.at[dst_v.at[b]], sem, add=True)
        pltpu.async_copy(ev_buf, den_sh.at[dst_v.at[b]], sem, add=True)

    def s_wait(usrc_v, ev_buf, sem):
        pltpu.make_async_copy(ud_hbm.at[pl.ds(0, _BLK)], usrc_v, sem).wait()
        pltpu.make_async_copy(nrm_hbm.at[pl.ds(0, _BLK)], ev_buf, sem).wait()

    bufs = [(udst0, usrc0, ev0), (udst1, usrc1, ev1), (udst2, usrc2, ev2)]
    gsems = [gsem0, gsem1, gsem2]
    ssems = [ssem0, ssem1, ssem2]

    g_issue(0, udst0, usrc0, gsem0)
    g_issue(1, udst1, usrc1, gsem1)

    def ring(ob, carry):
        for j in range(3):
            b = ob * 3 + j

            @pl.when(b < _NBLK)
            def _():
                udst_v, usrc_v, ev_buf = bufs[j]
                g_wait(udst_v, usrc_v, gsems[j])
                _compute_block(b, src_v, nrm_v, udst_v, usrc_v, ev_buf)
                s_issue(b, usrc_v, ev_buf, ssems[j])
                jn = (j + 2) % 3

                @pl.when(b <= _NBLK - 3)
                def _():
                    @pl.when(b >= 1)
                    def _():
                        s_wait(bufs[jn][1], bufs[jn][2], ssems[jn])
                    g_issue(b + 2, bufs[jn][0], bufs[jn][1], gsems[jn])
        return carry

    lax.fori_loop(0, (_NBLK + 2) // 3, ring, 0)

    s_wait(usrc2, ev2, ssem2)
    s_wait(usrc0, ev0, ssem0)
    s_wait(usrc1, ev1, ssem1)

    plsc.subcore_barrier()

    pltpu.sync_copy(den_sh.at[pl.ds(base, _STRIPE)],
                    den_out.at[c, 0, pl.ds(base, _STRIPE)])

    @pl.when(s < _NS - 1)
    def _():
        pltpu.sync_copy(num_sh.at[pl.ds(base, _STRIPE)],
                        num_out.at[c, pl.ds(base, _STRIPE)])

    @pl.when(s == _NS - 1)
    def _():
        pltpu.sync_copy(num_sh.at[pl.ds(base, _LAST)],
                        num_out.at[c, pl.ds(base, _LAST)])


_edge_conv = functools.partial(
    pl.kernel,
    compiler_params=pltpu.CompilerParams(needs_layout_passes=False, use_tc_tiling_on_sc=False),
    out_type=(
        jax.ShapeDtypeStruct((_NC, _N, _H), jnp.float32),
        jax.ShapeDtypeStruct((_NC, 1, _NP), jnp.float32),
    ),
    mesh=plsc.VectorSubcoreMesh(core_axis_name="c", subcore_axis_name="s",
                                num_cores=_NC, num_subcores=_NS),
    scratch_types=[
        pltpu.VMEM((_NBLK, _BLK), jnp.int32),
        pltpu.VMEM((_NBLK, _BLK), jnp.int32),
        pltpu.VMEM((_BLK, _H), jnp.float32),
        pltpu.VMEM((_BLK, _H), jnp.float32),
        pltpu.VMEM((_BLK,), jnp.float32),
        pltpu.VMEM((_BLK, _H), jnp.float32),
        pltpu.VMEM((_BLK, _H), jnp.float32),
        pltpu.VMEM((_BLK,), jnp.float32),
        pltpu.VMEM((_BLK, _H), jnp.float32),
        pltpu.VMEM((_BLK, _H), jnp.float32),
        pltpu.VMEM((_BLK,), jnp.float32),
        pltpu.VMEM((_N,), jnp.float32),
        pltpu.VMEM_SHARED((_N, _H), jnp.float32),
        pltpu.VMEM_SHARED((_NP,), jnp.float32),
        pltpu.SemaphoreType.DMA,
        pltpu.SemaphoreType.DMA,
        pltpu.SemaphoreType.DMA,
        pltpu.SemaphoreType.DMA,
        pltpu.SemaphoreType.DMA,
        pltpu.SemaphoreType.DMA,
    ],
)(_edge_body)


def kernel(x, edge_index, W1, b1, beta1, beta2, W2, b2):
    src = edge_index[0].reshape(_NW, _NBLK, _BLK)
    dst = edge_index[1].reshape(_NW, _NBLK, _BLK)

    u1, ud1, nrm1 = _dense_in(x, W1, b1, beta1)
    num1, den1 = _edge_conv(ud1, u1, nrm1.reshape(_N), src, dst)
    u2, ud2, nrm2 = _combine(num1, den1, beta2)
    num2, den2 = _edge_conv(ud2, u2, nrm2.reshape(_N), src, dst)
    return _final(num2, den2, W2, b2)


# speedup vs baseline: 1.0197x; 1.0197x over previous
import functools

import jax
import jax.numpy as jnp
from jax import lax
from jax.experimental import pallas as pl
from jax.experimental.pallas import tpu as pltpu
from jax.experimental.pallas import tpu_sc as plsc

_N = 10000
_E = 320000
_D = 128
_H = 64
_C = 40

_NC = 2
_NS = 16
_NW = _NC * _NS

_EPT = _E // _NW
_BLK = 80
_NBLK = _EPT // _BLK
_GRP = _BLK // 16

_STRIPE = 640
_LAST = _N - 15 * _STRIPE

_BR = 1000
_NP = _NS * _STRIPE


def _dense_in_body(beta_ref, x_ref, w_ref, b_ref, u_ref, ud_ref, nrm_ref):
    h = jnp.dot(x_ref[...], w_ref[...], preferred_element_type=jnp.float32)
    h = jnp.maximum(h + b_ref[...], 0.0)
    nrm = jnp.sqrt(jnp.sum(h * h, axis=1, keepdims=True))
    u = h / jnp.maximum(nrm, 1e-12)
    u_ref[...] = u
    ud_ref[...] = beta_ref[0, 0] * u
    nrm_ref[...] = nrm


def _dense_in(x, w1, b1, beta1):
    return pl.pallas_call(
        _dense_in_body,
        grid=(_N // _BR,),
        in_specs=[
            pl.BlockSpec((1, 1), lambda i: (0, 0)),
            pl.BlockSpec((_BR, _D), lambda i: (i, 0)),
            pl.BlockSpec((_D, _H), lambda i: (0, 0)),
            pl.BlockSpec((1, _H), lambda i: (0, 0)),
        ],
        out_specs=[
            pl.BlockSpec((_BR, _H), lambda i: (i, 0)),
            pl.BlockSpec((_BR, _H), lambda i: (i, 0)),
            pl.BlockSpec((_BR, 1), lambda i: (i, 0)),
        ],
        out_shape=[
            jax.ShapeDtypeStruct((_N, _H), jnp.float32),
            jax.ShapeDtypeStruct((_N, _H), jnp.float32),
            jax.ShapeDtypeStruct((_N, 1), jnp.float32),
        ],
    )(beta1.reshape(1, 1), x, w1, b1.reshape(1, _H))


def _combine_body(beta_ref, num_ref, den_ref, u_ref, ud_ref, nrm_ref):
    den = den_ref[0] + den_ref[1] + 1e-16
    h = (num_ref[0] + num_ref[1]) / den
    nrm = jnp.sqrt(jnp.sum(h * h, axis=1, keepdims=True))
    u = h / jnp.maximum(nrm, 1e-12)
    u_ref[...] = u
    ud_ref[...] = beta_ref[0, 0] * u
    nrm_ref[...] = nrm


def _combine(num, den, beta2):
    return pl.pallas_call(
        _combine_body,
        grid=(_N // _BR,),
        in_specs=[
            pl.BlockSpec((1, 1), lambda i: (0, 0)),
            pl.BlockSpec((_NC, _BR, _H), lambda i: (0, i, 0)),
            pl.BlockSpec((_NC, _BR, 1), lambda i: (0, i, 0)),
        ],
        out_specs=[
            pl.BlockSpec((_BR, _H), lambda i: (i, 0)),
            pl.BlockSpec((_BR, _H), lambda i: (i, 0)),
            pl.BlockSpec((_BR, 1), lambda i: (i, 0)),
        ],
        out_shape=[
            jax.ShapeDtypeStruct((_N, _H), jnp.float32),
            jax.ShapeDtypeStruct((_N, _H), jnp.float32),
            jax.ShapeDtypeStruct((_N, 1), jnp.float32),
        ],
    )(beta2.reshape(1, 1), num, den[:, 0, :_N].reshape(_NC, _N, 1))


def _final_body(num_ref, den_ref, w_ref, b_ref, out_ref):
    den = den_ref[0] + den_ref[1] + 1e-16
    h = (num_ref[0] + num_ref[1]) / den
    o = jnp.dot(h, w_ref[...], preferred_element_type=jnp.float32) + b_ref[...]
    m = jnp.max(o, axis=1, keepdims=True)
    lse = jnp.log(jnp.sum(jnp.exp(o - m), axis=1, keepdims=True))
    out_ref[...] = o - m - lse


def _final(num, den, w2, b2):
    return pl.pallas_call(
        _final_body,
        grid=(_N // _BR,),
        in_specs=[
            pl.BlockSpec((_NC, _BR, _H), lambda i: (0, i, 0)),
            pl.BlockSpec((_NC, _BR, 1), lambda i: (0, i, 0)),
            pl.BlockSpec((_H, _C), lambda i: (0, 0)),
            pl.BlockSpec((1, _C), lambda i: (0, 0)),
        ],
        out_specs=pl.BlockSpec((_BR, _C), lambda i: (i, 0)),
        out_shape=jax.ShapeDtypeStruct((_N, _C), jnp.float32),
    )(num, den[:, 0, :_N].reshape(_NC, _N, 1), w2, b2.reshape(1, _C))


def _compute_block(b, src_v, nrm_v, udst_v, usrc_v, ev_v):
    lanes = lax.iota(jnp.int32, 16)
    perms = [lanes ^ m for m in (1, 2, 4, 8)]
    gd = lax.GatherDimensionNumbers(offset_dims=(),
                                    collapsed_slice_dims=(0,),
                                    start_index_map=(0,))
    for g in range(_GRP):
        alpha_v = jnp.zeros((16,), jnp.float32)
        for e in range(16):
            r = g * 16 + e
            a = udst_v[r, 0:16] * usrc_v[r, 0:16]
            a = a + udst_v[r, 16:32] * usrc_v[r, 16:32]
            a = a + udst_v[r, 32:48] * usrc_v[r, 32:48]
            a = a + udst_v[r, 48:64] * usrc_v[r, 48:64]
            for p in perms:
                a = a + lax.gather(
                    a, p[:, None], gd, (1,),
                    mode=lax.GatherScatterMode.PROMISE_IN_BOUNDS)
            alpha_v = jnp.where(lanes == e, a, alpha_v)
        rr = pl.ds(g * 16, 16)
        ev = jnp.exp(alpha_v)
        ev_v[rr] = ev
        sidx = src_v[b, rr]
        fv = ev * plsc.load_gather(nrm_v, [sidx])
        for e in range(16):
            r = g * 16 + e
            f = fv[e]
            for k in range(4):
                sl = pl.ds(k * 16, 16)
                usrc_v[r, sl] = usrc_v[r, sl] * f


def _edge_body(ud_hbm, u_hbm, nrm_hbm, src_hbm, dst_hbm, num_out, den_out,
               src_v, dst_v, udst0, usrc0, ev0, udst1, usrc1, ev1,
               udst2, usrc2, ev2, nrm_v, num_sh, den_sh,
               gsem0, gsem1, gsem2, ssem0, ssem1, ssem2):
    c = lax.axis_index("c")
    s = lax.axis_index("s")
    wid = c * _NS + s

    z16 = jnp.zeros((16,), jnp.float32)
    for r in range(_BLK):
        for k in range(4):
            udst0[r, k * 16:(k + 1) * 16] = z16
    for k in range(_BLK // 16):
        ev0[k * 16:(k + 1) * 16] = z16

    base = s * _STRIPE

    for k in range(_STRIPE // _BLK):
        pltpu.sync_copy(ev0, den_sh.at[pl.ds(base + k * _BLK, _BLK)])

    @pl.when(s < _NS - 1)
    def _():
        for k in range(_STRIPE // _BLK):
            pltpu.sync_copy(udst0, num_sh.at[pl.ds(base + k * _BLK, _BLK)])

    @pl.when(s == _NS - 1)
    def _():
        for k in range(_LAST // _BLK):
            pltpu.sync_copy(udst0, num_sh.at[pl.ds(base + k * _BLK, _BLK)])

    pltpu.sync_copy(src_hbm.at[wid], src_v)
    pltpu.sync_copy(dst_hbm.at[wid], dst_v)
    pltpu.sync_copy(nrm_hbm, nrm_v)

    plsc.subcore_barrier()

    def g_issue(b, udst_v, usrc_v, sem):
        pltpu.async_copy(ud_hbm.at[dst_v.at[b]], udst_v, sem)
        pltpu.async_copy(u_hbm.at[src_v.at[b]], usrc_v, sem)

    def g_wait(udst_v, usrc_v, sem):
        pltpu.make_async_copy(ud_hbm.at[pl.ds(0, _BLK)], udst_v, sem).wait()
        pltpu.make_async_copy(u_hbm.at[pl.ds(0, _BLK)], usrc_v, sem).wait()

    def s_issue(b, usrc_v, ev_buf, sem):
        pltpu.async_copy(usrc_v, num_sh.at[dst_v.at[b]], sem, add=True)
        pltpu.async_copy(ev_buf, den_sh.at[dst_v.at[b]], sem, add=True)

    def s_wait(usrc_v, ev_buf, sem):
        pltpu.make_async_copy(ud_hbm.at[pl.ds(0, _BLK)], usrc_v, sem).wait()
        pltpu.make_async_copy(nrm_hbm.at[pl.ds(0, _BLK)], ev_buf, sem).wait()

    bufs = [(udst0, usrc0, ev0), (udst1, usrc1, ev1), (udst2, usrc2, ev2)]
    gsems = [gsem0, gsem1, gsem2]
    ssems = [ssem0, ssem1, ssem2]

    g_issue(0, udst0, usrc0, gsem0)
    g_issue(1, udst1, usrc1, gsem1)

    def ring(ob, carry):
        for j in range(3):
            b = ob * 3 + j

            @pl.when(b < _NBLK)
            def _():
                udst_v, usrc_v, ev_buf = bufs[j]
                g_wait(udst_v, usrc_v, gsems[j])
                _compute_block(b, src_v, nrm_v, udst_v, usrc_v, ev_buf)
                s_issue(b, usrc_v, ev_buf, ssems[j])
                jn = (j + 2) % 3

                @pl.when(b <= _NBLK - 3)
                def _():
                    @pl.when(b >= 1)
                    def _():
                        s_wait(bufs[jn][1], bufs[jn][2], ssems[jn])
                    g_issue(b + 2, bufs[jn][0], bufs[jn][1], gsems[jn])
        return carry

    lax.fori_loop(0, (_NBLK + 2) // 3, ring, 0)

    s_wait(usrc2, ev2, ssem2)
    s_wait(usrc0, ev0, ssem0)
    s_wait(usrc1, ev1, ssem1)

    plsc.subcore_barrier()

    pltpu.sync_copy(den_sh.at[pl.ds(base, _STRIPE)],
                    den_out.at[c, 0, pl.ds(base, _STRIPE)])

    @pl.when(s < _NS - 1)
    def _():
        pltpu.sync_copy(num_sh.at[pl.ds(base, _STRIPE)],
                        num_out.at[c, pl.ds(base, _STRIPE)])

    @pl.when(s == _NS - 1)
    def _():
        pltpu.sync_copy(num_sh.at[pl.ds(base, _LAST)],
                        num_out.at[c, pl.ds(base, _LAST)])


_edge_conv = functools.partial(
    pl.kernel,
    compiler_params=pltpu.CompilerParams(needs_layout_passes=False, use_tc_tiling_on_sc=False),
    out_type=(
        jax.ShapeDtypeStruct((_NC, _N, _H), jnp.float32),
        jax.ShapeDtypeStruct((_NC, 1, _NP), jnp.float32),
    ),
    mesh=plsc.VectorSubcoreMesh(core_axis_name="c", subcore_axis_name="s",
                                num_cores=_NC, num_subcores=_NS),
    scratch_types=[
        pltpu.VMEM((_NBLK, _BLK), jnp.int32),
        pltpu.VMEM((_NBLK, _BLK), jnp.int32),
        pltpu.VMEM((_BLK, _H), jnp.float32),
        pltpu.VMEM((_BLK, _H), jnp.float32),
        pltpu.VMEM((_BLK,), jnp.float32),
        pltpu.VMEM((_BLK, _H), jnp.float32),
        pltpu.VMEM((_BLK, _H), jnp.float32),
        pltpu.VMEM((_BLK,), jnp.float32),
        pltpu.VMEM((_BLK, _H), jnp.float32),
        pltpu.VMEM((_BLK, _H), jnp.float32),
        pltpu.VMEM((_BLK,), jnp.float32),
        pltpu.VMEM((_N,), jnp.float32),
        pltpu.VMEM_SHARED((_N, _H), jnp.float32),
        pltpu.VMEM_SHARED((_NP,), jnp.float32),
        pltpu.SemaphoreType.DMA,
        pltpu.SemaphoreType.DMA,
        pltpu.SemaphoreType.DMA,
        pltpu.SemaphoreType.DMA,
        pltpu.SemaphoreType.DMA,
        pltpu.SemaphoreType.DMA,
    ],
)(_edge_body)


def kernel(x, edge_index, W1, b1, beta1, beta2, W2, b2):
    src = edge_index[0].reshape(_NW, _NBLK, _BLK)
    dst = edge_index[1].reshape(_NW, _NBLK, _BLK)

    u1, ud1, nrm1 = _dense_in(x, W1, b1, beta1)
    num1, den1 = _edge_conv(ud1, u1, nrm1.reshape(_N), src, dst)
    u2, ud2, nrm2 = _combine(num1, den1, beta2)
    num2, den2 = _edge_conv(ud2, u2, nrm2.reshape(_N), src, dst)
    return _final(num2, den2, W2, b2)
